# baseline (speedup 1.0000x reference)
import functools

import jax
import jax.numpy as jnp
from jax import lax
from jax.experimental import pallas as pl
from jax.experimental.pallas import tpu as pltpu
from jax.experimental.pallas import tpu_sc as plsc

N_STIM = 21
N_DIM = 3
BATCH = 16384
NC, NS = 2, 16
NW = NC * NS
ROWS_PER_W = BATCH // NW
GROUPS = ROWS_PER_W // 16
SIM_STRIDE = 32
SIM_PAD = N_STIM * SIM_STRIDE


@functools.partial(
    pl.kernel,
    mesh=plsc.VectorSubcoreMesh(core_axis_name="c", subcore_axis_name="s"),
    out_type=jax.ShapeDtypeStruct((BATCH * 4,), jnp.float32),
    compiler_params=pltpu.CompilerParams(
        needs_layout_passes=False, use_tc_tiling_on_sc=False),
    scratch_types=[
        pltpu.VMEM((5, ROWS_PER_W), jnp.int32),
        pltpu.VMEM((N_DIM, SIM_STRIDE), jnp.float32),
        pltpu.VMEM((SIM_PAD,), jnp.float32),
        pltpu.VMEM((4 * ROWS_PER_W,), jnp.float32),
        pltpu.SemaphoreType.DMA,
    ],
)
def _rank_sc(stim_hbm, tab_hbm, out_hbm, idx_v, tab_v, sim_v, out_v, sem):
    wid = lax.axis_index("s") * NC + lax.axis_index("c")
    base = wid * ROWS_PER_W
    idx_dma = pltpu.async_copy(
        stim_hbm.at[:, pl.ds(base, ROWS_PER_W)], idx_v, sem)
    pltpu.sync_copy(tab_hbm, tab_v)

    xj = [[tab_v[d, pl.ds(h * 16, 16)] for h in range(2)]
          for d in range(N_DIM)]

    def build_sim(i, carry):
        ivec = jnp.zeros((16,), jnp.int32) + i
        ti = [plsc.load_gather(tab_v, [jnp.full((16,), d, jnp.int32), ivec])
              for d in range(N_DIM)]
        for h in range(2):
            d2 = jnp.zeros((16,), jnp.float32)
            for d in range(N_DIM):
                diff = ti[d] - xj[d][h]
                d2 = d2 + diff * diff
            bits = lax.bitcast_convert_type(d2, jnp.int32)
            y = lax.bitcast_convert_type(
                jnp.int32(0x5F3759DF) - lax.shift_right_logical(bits, 1),
                jnp.float32)
            for _ in range(3):
                y = y * (1.5 - 0.5 * d2 * y * y)
            dist = d2 * y
            sim_v[pl.ds(i * SIM_STRIDE + h * 16, 16)] = jnp.exp(-10.0 * dist)
        return carry

    lax.fori_loop(0, N_STIM, build_sim, None)
    idx_dma.wait()

    def do_group(g, carry):
        off = g * 16
        q = idx_v[0, pl.ds(off, 16)]
        qbase = q * SIM_STRIDE
        s = []
        for k in range(4):
            r = idx_v[k + 1, pl.ds(off, 16)]
            s.append(plsc.load_gather(sim_v, [qbase + r]))
        inv = 1.0 / (s[0] + s[1] + s[2] + s[3])
        obase = (g // 8) * 512 + (g % 8) * 16
        for k in range(4):
            out_v[pl.ds(obase + k * 128, 16)] = s[k] * inv
        return carry

    lax.fori_loop(0, GROUPS, do_group, None)

    pltpu.sync_copy(out_v, out_hbm.at[pl.ds(wid * 4 * ROWS_PER_W,
                                            4 * ROWS_PER_W)])


def kernel(given4rank1_stimulus_set, table):
    stim_cm = jnp.transpose(given4rank1_stimulus_set)
    tab_cm = jnp.pad(jnp.transpose(table), ((0, 0), (0, SIM_STRIDE - N_STIM)))
    out_flat = _rank_sc(stim_cm, tab_cm)
    out3 = jnp.reshape(out_flat, (BATCH // 128, 4, 128))
    return jnp.reshape(jnp.transpose(out3, (0, 2, 1)), (BATCH, 4))


# speedup vs baseline: 1.0418x; 1.0418x over previous
import functools

import jax
import jax.numpy as jnp
from jax import lax
from jax.experimental import pallas as pl
from jax.experimental.pallas import tpu as pltpu
from jax.experimental.pallas import tpu_sc as plsc

N_STIM = 21
N_DIM = 3
BATCH = 16384
NC, NS = 1, 16
NW = NC * NS
ROWS_PER_W = BATCH // NW
GROUPS = ROWS_PER_W // 16
SIM_STRIDE = 32
SIM_PAD = N_STIM * SIM_STRIDE


@functools.partial(
    pl.kernel,
    mesh=plsc.VectorSubcoreMesh(core_axis_name="c", subcore_axis_name="s", num_cores=1),
    out_type=jax.ShapeDtypeStruct((BATCH * 4,), jnp.float32),
    compiler_params=pltpu.CompilerParams(
        needs_layout_passes=False, use_tc_tiling_on_sc=False),
    scratch_types=[
        pltpu.VMEM((5, ROWS_PER_W), jnp.int32),
        pltpu.VMEM((N_DIM, SIM_STRIDE), jnp.float32),
        pltpu.VMEM((SIM_PAD,), jnp.float32),
        pltpu.VMEM((4 * ROWS_PER_W,), jnp.float32),
        pltpu.SemaphoreType.DMA,
    ],
)
def _rank_sc(stim_hbm, tab_hbm, out_hbm, idx_v, tab_v, sim_v, out_v, sem):
    wid = lax.axis_index("s") * NC + lax.axis_index("c")
    base = wid * ROWS_PER_W
    idx_dma = pltpu.async_copy(
        stim_hbm.at[:, pl.ds(base, ROWS_PER_W)], idx_v, sem)
    pltpu.sync_copy(tab_hbm, tab_v)

    xj = [[tab_v[d, pl.ds(h * 16, 16)] for h in range(2)]
          for d in range(N_DIM)]

    def build_sim(i, carry):
        ivec = jnp.zeros((16,), jnp.int32) + i
        ti = [plsc.load_gather(tab_v, [jnp.full((16,), d, jnp.int32), ivec])
              for d in range(N_DIM)]
        for h in range(2):
            d2 = jnp.zeros((16,), jnp.float32)
            for d in range(N_DIM):
                diff = ti[d] - xj[d][h]
                d2 = d2 + diff * diff
            bits = lax.bitcast_convert_type(d2, jnp.int32)
            y = lax.bitcast_convert_type(
                jnp.int32(0x5F3759DF) - lax.shift_right_logical(bits, 1),
                jnp.float32)
            for _ in range(3):
                y = y * (1.5 - 0.5 * d2 * y * y)
            dist = d2 * y
            sim_v[pl.ds(i * SIM_STRIDE + h * 16, 16)] = jnp.exp(-10.0 * dist)
        return carry

    lax.fori_loop(0, N_STIM, build_sim, None)
    idx_dma.wait()

    def do_group(g, carry):
        off = g * 16
        q = idx_v[0, pl.ds(off, 16)]
        qbase = q * SIM_STRIDE
        s = []
        for k in range(4):
            r = idx_v[k + 1, pl.ds(off, 16)]
            s.append(plsc.load_gather(sim_v, [qbase + r]))
        inv = 1.0 / (s[0] + s[1] + s[2] + s[3])
        obase = (g // 8) * 512 + (g % 8) * 16
        for k in range(4):
            out_v[pl.ds(obase + k * 128, 16)] = s[k] * inv
        return carry

    lax.fori_loop(0, GROUPS, do_group, None)

    pltpu.sync_copy(out_v, out_hbm.at[pl.ds(wid * 4 * ROWS_PER_W,
                                            4 * ROWS_PER_W)])


def kernel(given4rank1_stimulus_set, table):
    stim_cm = jnp.transpose(given4rank1_stimulus_set)
    tab_cm = jnp.pad(jnp.transpose(table), ((0, 0), (0, SIM_STRIDE - N_STIM)))
    out_flat = _rank_sc(stim_cm, tab_cm)
    out3 = jnp.reshape(out_flat, (BATCH // 128, 4, 128))
    return jnp.reshape(jnp.transpose(out3, (0, 2, 1)), (BATCH, 4))


# speedup vs baseline: 1.0488x; 1.0067x over previous
import functools

import jax
import jax.numpy as jnp
from jax import lax
from jax.experimental import pallas as pl
from jax.experimental.pallas import tpu as pltpu
from jax.experimental.pallas import tpu_sc as plsc

N_STIM = 21
N_DIM = 3
BATCH = 16384
NC, NS = 1, 16
NW = NC * NS
ROWS_PER_W = BATCH // NW
GROUPS = ROWS_PER_W // 16
SIM_STRIDE = 32
SIM_PAD = N_STIM * SIM_STRIDE


@functools.partial(
    pl.kernel,
    mesh=plsc.VectorSubcoreMesh(core_axis_name="c", subcore_axis_name="s", num_cores=1),
    out_type=jax.ShapeDtypeStruct((BATCH * 4,), jnp.float32),
    compiler_params=pltpu.CompilerParams(
        needs_layout_passes=False, use_tc_tiling_on_sc=True),
    scratch_types=[
        pltpu.VMEM((5, ROWS_PER_W), jnp.int32),
        pltpu.VMEM((N_DIM, SIM_STRIDE), jnp.float32),
        pltpu.VMEM((SIM_PAD,), jnp.float32),
        pltpu.VMEM((4 * ROWS_PER_W,), jnp.float32),
        pltpu.SemaphoreType.DMA,
    ],
)
def _rank_sc(stim_hbm, tab_hbm, out_hbm, idx_v, tab_v, sim_v, out_v, sem):
    wid = lax.axis_index("s") * NC + lax.axis_index("c")
    base = wid * ROWS_PER_W
    idx_dma = pltpu.async_copy(
        stim_hbm.at[:, pl.ds(base, ROWS_PER_W)], idx_v, sem)
    pltpu.sync_copy(tab_hbm, tab_v)

    xj = [[tab_v[d, pl.ds(h * 16, 16)] for h in range(2)]
          for d in range(N_DIM)]

    def build_sim(i, carry):
        ivec = jnp.zeros((16,), jnp.int32) + i
        ti = [plsc.load_gather(tab_v, [jnp.full((16,), d, jnp.int32), ivec])
              for d in range(N_DIM)]
        for h in range(2):
            d2 = jnp.zeros((16,), jnp.float32)
            for d in range(N_DIM):
                diff = ti[d] - xj[d][h]
                d2 = d2 + diff * diff
            bits = lax.bitcast_convert_type(d2, jnp.int32)
            y = lax.bitcast_convert_type(
                jnp.int32(0x5F3759DF) - lax.shift_right_logical(bits, 1),
                jnp.float32)
            for _ in range(3):
                y = y * (1.5 - 0.5 * d2 * y * y)
            dist = d2 * y
            sim_v[pl.ds(i * SIM_STRIDE + h * 16, 16)] = jnp.exp(-10.0 * dist)
        return carry

    lax.fori_loop(0, N_STIM, build_sim, None)
    idx_dma.wait()

    def do_group(g, carry):
        off = g * 16
        q = idx_v[0, pl.ds(off, 16)]
        qbase = q * SIM_STRIDE
        s = []
        for k in range(4):
            r = idx_v[k + 1, pl.ds(off, 16)]
            s.append(plsc.load_gather(sim_v, [qbase + r]))
        inv = 1.0 / (s[0] + s[1] + s[2] + s[3])
        obase = (g // 8) * 512 + (g % 8) * 16
        for k in range(4):
            out_v[pl.ds(obase + k * 128, 16)] = s[k] * inv
        return carry

    lax.fori_loop(0, GROUPS, do_group, None)

    pltpu.sync_copy(out_v, out_hbm.at[pl.ds(wid * 4 * ROWS_PER_W,
                                            4 * ROWS_PER_W)])


def kernel(given4rank1_stimulus_set, table):
    stim_cm = jnp.transpose(given4rank1_stimulus_set)
    tab_cm = jnp.pad(jnp.transpose(table), ((0, 0), (0, SIM_STRIDE - N_STIM)))
    out_flat = _rank_sc(stim_cm, tab_cm)
    out3 = jnp.reshape(out_flat, (BATCH // 128, 4, 128))
    return jnp.reshape(jnp.transpose(out3, (0, 2, 1)), (BATCH, 4))
